# Initial kernel scaffold; baseline (speedup 1.0000x reference)
import jax
import jax.numpy as jnp
from jax.experimental import pallas as pl


def kernel(index_vectors, edge_index, w_zeroth, w_fst):
    raise NotImplementedError("write your pallas kernel here")


# speedup vs baseline: 3.4987x; 3.4987x over previous
import functools

import jax
import jax.numpy as jnp
from jax import lax
from jax.experimental import pallas as pl
from jax.experimental.pallas import tpu as pltpu
from jax.experimental.pallas import tpu_sc as plsc

N = 10000
E = 320000
D = 128

NC = 2
NS = 16
NW = NC * NS

CHUNK = 128
CH = 80
SLAB = 16
NSLAB = CH // SLAB
E_PAD = NW * CH * CHUNK
DUMP = N
N_PAD = 10112
ROWS_PER_TILE = N_PAD // NS


def _sc_body(x_hbm, col_hbm, row_hbm, acc_hbm,
             col_v, row_v, rows_a, rows_b, acc_s, sem_a, sem_b):
    cid = lax.axis_index("c")
    sid = lax.axis_index("s")
    wid = sid * NC + cid

    zeros16 = jnp.zeros((16,), jnp.float32)

    def zloop(i, carry):
        r = i // (D // 16)
        c = (i % (D // 16)) * 16
        rows_a[r, pl.ds(c, 16)] = zeros16
        return carry

    lax.fori_loop(0, CHUNK * (D // 16), zloop, 0)
    base = sid * ROWS_PER_TILE
    for k in range(ROWS_PER_TILE // CHUNK):
        pltpu.sync_copy(rows_a, acc_s.at[pl.ds(base + k * CHUNK, CHUNK)])
    rem = ROWS_PER_TILE % CHUNK
    if rem:
        off = ROWS_PER_TILE - rem
        pltpu.sync_copy(rows_a.at[pl.ds(0, rem)],
                        acc_s.at[pl.ds(base + off, rem)])
    plsc.subcore_barrier()

    for s in range(NSLAB):
        pltpu.sync_copy(col_hbm.at[wid, pl.ds(s * SLAB, SLAB)], col_v)
        pltpu.sync_copy(row_hbm.at[wid, pl.ds(s * SLAB, SLAB)], row_v)
        pltpu.async_copy(x_hbm.at[col_v.at[0]], rows_a, sem_a)
        pltpu.async_copy(x_hbm.at[col_v.at[1]], rows_b, sem_b)

        def mloop(j2, carry):
            j = j2 * 2
            pltpu.make_async_copy(x_hbm.at[col_v.at[j]], rows_a, sem_a).wait()
            pltpu.sync_copy(rows_a, acc_s.at[row_v.at[j]], add=True)
            pltpu.async_copy(x_hbm.at[col_v.at[j + 2]], rows_a, sem_a)
            pltpu.make_async_copy(x_hbm.at[col_v.at[j + 1]], rows_b, sem_b).wait()
            pltpu.sync_copy(rows_b, acc_s.at[row_v.at[j + 1]], add=True)
            pltpu.async_copy(x_hbm.at[col_v.at[j + 3]], rows_b, sem_b)
            return carry

        lax.fori_loop(0, SLAB // 2 - 1, mloop, 0)
        j = SLAB - 2
        pltpu.make_async_copy(x_hbm.at[col_v.at[j]], rows_a, sem_a).wait()
        pltpu.sync_copy(rows_a, acc_s.at[row_v.at[j]], add=True)
        pltpu.make_async_copy(x_hbm.at[col_v.at[j + 1]], rows_b, sem_b).wait()
        pltpu.sync_copy(rows_b, acc_s.at[row_v.at[j + 1]], add=True)

    plsc.subcore_barrier()
    pltpu.sync_copy(acc_s.at[pl.ds(base, ROWS_PER_TILE)],
                    acc_hbm.at[cid, pl.ds(base, ROWS_PER_TILE)])


_sc_spmm = functools.partial(
    pl.kernel,
    out_type=jax.ShapeDtypeStruct((NC, N_PAD, D), jnp.float32),
    mesh=plsc.VectorSubcoreMesh(core_axis_name="c", subcore_axis_name="s"),
    scratch_types=[
        pltpu.VMEM((SLAB, CHUNK), jnp.int32),
        pltpu.VMEM((SLAB, CHUNK), jnp.int32),
        pltpu.VMEM((CHUNK, D), jnp.float32),
        pltpu.VMEM((CHUNK, D), jnp.float32),
        pltpu.VMEM_SHARED((N_PAD, D), jnp.float32),
        pltpu.SemaphoreType.DMA,
        pltpu.SemaphoreType.DMA,
    ],
)(_sc_body)


def _tc_finalize_body(x_ref, acc_ref, wz_ref, wf_ref, o_ref):
    wz = jnp.maximum(wz_ref[0, 0], 0.0)
    wf = jnp.maximum(wf_ref[0, 0], 0.0)
    tot = wz + wf + 1e-6
    ctx = acc_ref[0] + acc_ref[1]
    o_ref[...] = (wz / tot) * x_ref[...] + (wf / tot) * ctx


def _tc_finalize(x, acc, wz, wf):
    br = 1000
    grid = (N // br,)
    return pl.pallas_call(
        _tc_finalize_body,
        out_shape=jax.ShapeDtypeStruct((N, D), jnp.float32),
        grid=grid,
        in_specs=[
            pl.BlockSpec((br, D), lambda i: (i, 0)),
            pl.BlockSpec((NC, br, D), lambda i: (0, i, 0)),
            pl.BlockSpec((1, 1), lambda i: (0, 0)),
            pl.BlockSpec((1, 1), lambda i: (0, 0)),
        ],
        out_specs=pl.BlockSpec((br, D), lambda i: (i, 0)),
    )(x, acc, wz, wf)


def kernel(index_vectors, edge_index, w_zeroth, w_fst):
    row = edge_index[0]
    col = edge_index[1]
    pad = E_PAD - E
    col_p = jnp.concatenate([col, jnp.zeros((pad,), jnp.int32)])
    row_p = jnp.concatenate([row, jnp.full((pad,), DUMP, jnp.int32)])
    col_r = col_p.reshape(NW, CH, CHUNK)
    row_r = row_p.reshape(NW, CH, CHUNK)
    acc = _sc_spmm(index_vectors, col_r, row_r)
    return _tc_finalize(index_vectors, acc, w_zeroth, w_fst)


# speedup vs baseline: 3.7758x; 1.0792x over previous
import functools

import jax
import jax.numpy as jnp
from jax import lax
from jax.experimental import pallas as pl
from jax.experimental.pallas import tpu as pltpu
from jax.experimental.pallas import tpu_sc as plsc

N = 10000
E = 320000
D = 128

NC = 2
NS = 16
NW = NC * NS

CHUNK = 128
CH0 = 128
CH1 = 32
SLAB = 16
E_PAD = NS * (CH0 + CH1) * CHUNK
DUMP = N
N_PAD = 10112
ROWS_PER_TILE = N_PAD // NS


def _sc_body(x_hbm, col0_hbm, row0_hbm, col1_hbm, row1_hbm, acc_hbm,
             col_v, row_v, rows_a, rows_b, acc_s, sem_a, sem_b):
    cid = lax.axis_index("c")
    sid = lax.axis_index("s")

    zeros16 = jnp.zeros((16,), jnp.float32)

    def zloop(i, carry):
        r = i // (D // 16)
        c = (i % (D // 16)) * 16
        rows_a[r, pl.ds(c, 16)] = zeros16
        return carry

    lax.fori_loop(0, CHUNK * (D // 16), zloop, 0)
    base = sid * ROWS_PER_TILE
    for k in range(ROWS_PER_TILE // CHUNK):
        pltpu.sync_copy(rows_a, acc_s.at[pl.ds(base + k * CHUNK, CHUNK)])
    rem = ROWS_PER_TILE % CHUNK
    if rem:
        off = ROWS_PER_TILE - rem
        pltpu.sync_copy(rows_a.at[pl.ds(0, rem)],
                        acc_s.at[pl.ds(base + off, rem)])
    plsc.subcore_barrier()

    def run_edges(col_hbm, row_hbm, n_slabs):
        for s in range(n_slabs):
            pltpu.sync_copy(col_hbm.at[sid, pl.ds(s * SLAB, SLAB)], col_v)
            pltpu.sync_copy(row_hbm.at[sid, pl.ds(s * SLAB, SLAB)], row_v)
            pltpu.async_copy(x_hbm.at[col_v.at[0]], rows_a, sem_a)
            pltpu.async_copy(x_hbm.at[col_v.at[1]], rows_b, sem_b)

            def mloop(j2, carry):
                j = j2 * 2
                pltpu.make_async_copy(x_hbm.at[col_v.at[j]], rows_a, sem_a).wait()
                pltpu.sync_copy(rows_a, acc_s.at[row_v.at[j]], add=True)
                pltpu.async_copy(x_hbm.at[col_v.at[j + 2]], rows_a, sem_a)
                pltpu.make_async_copy(x_hbm.at[col_v.at[j + 1]], rows_b, sem_b).wait()
                pltpu.sync_copy(rows_b, acc_s.at[row_v.at[j + 1]], add=True)
                pltpu.async_copy(x_hbm.at[col_v.at[j + 3]], rows_b, sem_b)
                return carry

            lax.fori_loop(0, SLAB // 2 - 1, mloop, 0)
            j = SLAB - 2
            pltpu.make_async_copy(x_hbm.at[col_v.at[j]], rows_a, sem_a).wait()
            pltpu.sync_copy(rows_a, acc_s.at[row_v.at[j]], add=True)
            pltpu.make_async_copy(x_hbm.at[col_v.at[j + 1]], rows_b, sem_b).wait()
            pltpu.sync_copy(rows_b, acc_s.at[row_v.at[j + 1]], add=True)

    @pl.when(cid == 0)
    def _():
        run_edges(col0_hbm, row0_hbm, CH0 // SLAB)

    @pl.when(cid == 1)
    def _():
        run_edges(col1_hbm, row1_hbm, CH1 // SLAB)

    plsc.subcore_barrier()
    pltpu.sync_copy(acc_s.at[pl.ds(base, ROWS_PER_TILE)],
                    acc_hbm.at[cid, pl.ds(base, ROWS_PER_TILE)])


_sc_spmm = functools.partial(
    pl.kernel,
    out_type=jax.ShapeDtypeStruct((NC, N_PAD, D), jnp.float32),
    mesh=plsc.VectorSubcoreMesh(core_axis_name="c", subcore_axis_name="s"),
    scratch_types=[
        pltpu.VMEM((SLAB, CHUNK), jnp.int32),
        pltpu.VMEM((SLAB, CHUNK), jnp.int32),
        pltpu.VMEM((CHUNK, D), jnp.float32),
        pltpu.VMEM((CHUNK, D), jnp.float32),
        pltpu.VMEM_SHARED((N_PAD, D), jnp.float32),
        pltpu.SemaphoreType.DMA,
        pltpu.SemaphoreType.DMA,
    ],
)(_sc_body)


def _tc_finalize_body(x_ref, acc_ref, wz_ref, wf_ref, o_ref):
    wz = jnp.maximum(wz_ref[0, 0], 0.0)
    wf = jnp.maximum(wf_ref[0, 0], 0.0)
    tot = wz + wf + 1e-6
    ctx = acc_ref[0] + acc_ref[1]
    o_ref[...] = (wz / tot) * x_ref[...] + (wf / tot) * ctx


def _tc_finalize(x, acc, wz, wf):
    br = 1000
    grid = (N // br,)
    return pl.pallas_call(
        _tc_finalize_body,
        out_shape=jax.ShapeDtypeStruct((N, D), jnp.float32),
        grid=grid,
        in_specs=[
            pl.BlockSpec((br, D), lambda i: (i, 0)),
            pl.BlockSpec((NC, br, D), lambda i: (0, i, 0)),
            pl.BlockSpec((1, 1), lambda i: (0, 0)),
            pl.BlockSpec((1, 1), lambda i: (0, 0)),
        ],
        out_specs=pl.BlockSpec((br, D), lambda i: (i, 0)),
    )(x, acc, wz, wf)


def kernel(index_vectors, edge_index, w_zeroth, w_fst):
    row = edge_index[0]
    col = edge_index[1]
    pad = E_PAD - E
    col_p = jnp.concatenate([col, jnp.zeros((pad,), jnp.int32)])
    row_p = jnp.concatenate([row, jnp.full((pad,), DUMP, jnp.int32)])
    e0 = NS * CH0 * CHUNK
    col0 = col_p[:e0].reshape(NS, CH0, CHUNK)
    row0 = row_p[:e0].reshape(NS, CH0, CHUNK)
    col1 = col_p[e0:].reshape(NS, CH1, CHUNK)
    row1 = row_p[e0:].reshape(NS, CH1, CHUNK)
    acc = _sc_spmm(index_vectors, col0, row0, col1, row1)
    return _tc_finalize(index_vectors, acc, w_zeroth, w_fst)
